# Initial kernel scaffold; baseline (speedup 1.0000x reference)
import jax
import jax.numpy as jnp
from jax.experimental import pallas as pl


def kernel(x, edge_index, edge_index_aux, W1, b1, Wk1_1, bk1_1, Wk1_2, bk1_2, Wk2_1, bk2_1, Wk2_2, bk2_2, Wk3_1, bk3_1, Wk3_2, bk3_2, Wr0, br0, Wr1, br1, Wr2, br2, Wr3, br3, w0, w1, w2, w3):
    raise NotImplementedError("write your pallas kernel here")


# speedup vs baseline: 41.5604x; 41.5604x over previous
import functools

import jax
import jax.numpy as jnp
from jax import lax
from jax.experimental import pallas as pl
from jax.experimental.pallas import tpu as pltpu
from jax.experimental.pallas import tpu_sc as plsc

_NB = 8
_ROW_BLK = 1024
_SC_PARAMS = pltpu.CompilerParams(use_tc_tiling_on_sc=False)


def _sc_degree(e1, e2, npad, n_sub, cpt):
    rpt = npad // n_sub
    nbg = cpt // _NB
    mesh = plsc.VectorSubcoreMesh(core_axis_name="c", subcore_axis_name="s")
    zeros = jnp.zeros((npad,), jnp.float32)

    @functools.partial(
        pl.kernel,
        out_type=[jax.ShapeDtypeStruct((npad,), jnp.float32)] * 2,
        mesh=mesh,
        compiler_params=_SC_PARAMS,
        scratch_types=[
            pltpu.VMEM_SHARED((npad,), jnp.float32),
            pltpu.VMEM((_NB, 128), jnp.int32),
            pltpu.VMEM((128,), jnp.float32),
            pltpu.VMEM((rpt,), jnp.float32),
            pltpu.SemaphoreType.DMA,
        ],
    )
    def knl(e1_h, e2_h, z_h, d1_h, d2_h, acc, dstb, onesb, vbuf, ssem):
        c = lax.axis_index("c")
        s = lax.axis_index("s")
        for i in range(8):
            onesb[pl.ds(i * 16, 16)] = jnp.full((16,), 1.0, jnp.float32)
        pltpu.sync_copy(z_h.at[pl.ds(s * rpt, rpt)], vbuf)
        pltpu.sync_copy(vbuf, acc.at[pl.ds(s * rpt, rpt)])
        plsc.subcore_barrier()

        def run(e_h, out_h):
            base = s * cpt

            def outer(g, carry):
                gb = base + g * _NB
                pltpu.sync_copy(e_h.at[1, pl.ds(gb, _NB)], dstb)
                cps = [pltpu.async_copy(onesb, acc.at[dstb.at[b]], ssem, add=True)
                       for b in range(_NB)]
                for cp in cps:
                    cp.wait()
                return carry

            lax.fori_loop(0, nbg, outer, 0)
            plsc.subcore_barrier()
            pltpu.sync_copy(acc.at[pl.ds(s * rpt, rpt)], vbuf)
            pltpu.sync_copy(vbuf, out_h.at[pl.ds(s * rpt, rpt)])

        @pl.when(c == 0)
        def _():
            run(e1_h, d1_h)

        @pl.when(c == 1)
        def _():
            run(e2_h, d2_h)

    return knl(e1, e2, zeros)


def _sc_stage(e1, e2, t1, t2, k, npad, n_sub, cpt):
    rpt = npad // n_sub
    nbg = cpt // _NB
    mesh = plsc.VectorSubcoreMesh(core_axis_name="c", subcore_axis_name="s")
    zeros = jnp.zeros((npad, k), jnp.float32)

    @functools.partial(
        pl.kernel,
        out_type=[jax.ShapeDtypeStruct((npad, k), jnp.float32)] * 2,
        mesh=mesh,
        compiler_params=_SC_PARAMS,
        scratch_types=[
            pltpu.VMEM_SHARED((npad, k), jnp.float32),
            pltpu.VMEM_SHARED((npad, k), jnp.float32),
            pltpu.VMEM((_NB, 128), jnp.int32),
            pltpu.VMEM((_NB, 128), jnp.int32),
            pltpu.VMEM((_NB, 128, k), jnp.float32),
            pltpu.VMEM((rpt, k), jnp.float32),
            pltpu.SemaphoreType.DMA,
            pltpu.SemaphoreType.DMA,
        ],
    )
    def knl(e1_h, e2_h, t1_h, t2_h, z_h, y1_h, y2_h,
            acc, tbl, srcb, dstb, rows, vbuf, gsem, ssem):
        c = lax.axis_index("c")
        s = lax.axis_index("s")

        def run(e_h, t_h, out_h):
            sl = pl.ds(s * rpt, rpt)
            pltpu.sync_copy(t_h.at[sl], vbuf)
            pltpu.sync_copy(vbuf, tbl.at[sl])
            pltpu.sync_copy(z_h.at[sl], vbuf)
            pltpu.sync_copy(vbuf, acc.at[sl])
            plsc.subcore_barrier()
            base = s * cpt

            def outer(g, carry):
                gb = base + g * _NB
                pltpu.sync_copy(e_h.at[0, pl.ds(gb, _NB)], srcb)
                pltpu.sync_copy(e_h.at[1, pl.ds(gb, _NB)], dstb)
                gcps = [pltpu.async_copy(tbl.at[srcb.at[b]], rows.at[b], gsem)
                        for b in range(_NB)]
                for cp in gcps:
                    cp.wait()
                scps = [pltpu.async_copy(rows.at[b], acc.at[dstb.at[b]], ssem, add=True)
                        for b in range(_NB)]
                for cp in scps:
                    cp.wait()
                return carry

            lax.fori_loop(0, nbg, outer, 0)
            plsc.subcore_barrier()
            pltpu.sync_copy(acc.at[sl], vbuf)
            pltpu.sync_copy(vbuf, out_h.at[sl])

        @pl.when(c == 0)
        def _():
            run(e1_h, t1_h, y1_h)

        @pl.when(c == 1)
        def _():
            run(e2_h, t2_h, y2_h)

    return knl(e1, e2, t1, t2, zeros)


def _full(shape):
    return pl.BlockSpec(shape, lambda i: (0,) * len(shape))


def _rows(k):
    return pl.BlockSpec((_ROW_BLK, k), lambda i: (i, 0))


def _tc_dense(x, W1, b1, C, deg1, deg2, npad):
    n = x.shape[0]
    d_in = x.shape[1]
    h = W1.shape[1]
    grid = (npad + _ROW_BLK - 1) // _ROW_BLK

    def body(x_r, w1_r, b1_r, c_r, g1_r, g2_r, t1_o, t2_o, zr_o, d1_o, d2_o):
        r0 = jnp.maximum(jnp.dot(x_r[...], w1_r[...],
                                 preferred_element_type=jnp.float32) + b1_r[...], 0.0)
        z = jnp.dot(r0, c_r[...], preferred_element_type=jnp.float32)
        d1 = jnp.where(g1_r[...] > 0,
                       lax.rsqrt(jnp.maximum(g1_r[...], 1e-12)), 0.0)
        d2 = jnp.where(g2_r[...] > 0,
                       lax.rsqrt(jnp.maximum(g2_r[...], 1e-12)), 0.0)
        pad = jnp.zeros((z.shape[0], 2), jnp.float32)
        t1_o[...] = jnp.concatenate([z[:, 0:6] * d1, pad], axis=1)
        t2_o[...] = jnp.concatenate([z[:, 6:12] * d2, pad], axis=1)
        zr_o[...] = z[:, 12:15]
        d1_o[...] = d1
        d2_o[...] = d2

    return pl.pallas_call(
        body,
        grid=(grid,),
        in_specs=[_rows(d_in), _full((d_in, h)), _full((1, h)), _full((h, 15)),
                  _rows(1), _rows(1)],
        out_specs=[_rows(8), _rows(8), _rows(3), _rows(1), _rows(1)],
        out_shape=[jax.ShapeDtypeStruct((npad, 8), jnp.float32),
                   jax.ShapeDtypeStruct((npad, 8), jnp.float32),
                   jax.ShapeDtypeStruct((npad, 3), jnp.float32),
                   jax.ShapeDtypeStruct((npad, 1), jnp.float32),
                   jax.ShapeDtypeStruct((npad, 1), jnp.float32)],
    )(x, W1, b1, C, deg1, deg2)


def _tc_prep2(y1raw, y2raw, d1, d2, zr, consts, npad):
    grid = (npad + _ROW_BLK - 1) // _ROW_BLK

    def body(y1_r, y2_r, d1_r, d2_r, zr_r, c_r, u1_o, u2_o, s3_o):
        y1 = y1_r[...] * d1_r[...]
        y2 = y2_r[...] * d2_r[...]
        p = y1[:, 0:4] + y2[:, 0:4] + c_r[0, 0:4]
        pad = jnp.zeros((p.shape[0], 6), jnp.float32)
        u1_o[...] = jnp.concatenate(
            [jnp.concatenate([p[:, 0:1], p[:, 2:3]], axis=1) * d1_r[...], pad], axis=1)
        u2_o[...] = jnp.concatenate(
            [jnp.concatenate([p[:, 1:2], p[:, 3:4]], axis=1) * d2_r[...], pad], axis=1)
        s3_o[...] = zr_r[:, 0:2] + y1[:, 4:6] + y2[:, 4:6] + c_r[0, 4:6]

    return pl.pallas_call(
        body,
        grid=(grid,),
        in_specs=[_rows(8), _rows(8), _rows(1), _rows(1), _rows(3), _full((1, 8))],
        out_specs=[_rows(8), _rows(8), _rows(2)],
        out_shape=[jax.ShapeDtypeStruct((npad, 8), jnp.float32),
                   jax.ShapeDtypeStruct((npad, 8), jnp.float32),
                   jax.ShapeDtypeStruct((npad, 2), jnp.float32)],
    )(y1raw, y2raw, d1, d2, zr, consts)


def _tc_prep3(q1raw, q2raw, d1, d2, s3p, consts, npad):
    grid = (npad + _ROW_BLK - 1) // _ROW_BLK

    def body(q1_r, q2_r, d1_r, d2_r, s3_r, c_r, g1_o, g2_o):
        f = (s3_r[...] + q1_r[:, 0:2] * d1_r[...] + q2_r[:, 0:2] * d2_r[...]
             + c_r[0, :])
        pad = jnp.zeros((f.shape[0], 7), jnp.float32)
        g1_o[...] = jnp.concatenate([f[:, 0:1] * d1_r[...], pad], axis=1)
        g2_o[...] = jnp.concatenate([f[:, 1:2] * d2_r[...], pad], axis=1)

    return pl.pallas_call(
        body,
        grid=(grid,),
        in_specs=[_rows(8), _rows(8), _rows(1), _rows(1), _rows(2), _full((1, 2))],
        out_specs=[_rows(8), _rows(8)],
        out_shape=[jax.ShapeDtypeStruct((npad, 8), jnp.float32)] * 2,
    )(q1raw, q2raw, d1, d2, s3p, consts)


def _tc_final(h1raw, h2raw, d1, d2, zr, kout, npad):
    grid = (npad + _ROW_BLK - 1) // _ROW_BLK

    def body(h1_r, h2_r, d1_r, d2_r, zr_r, k_r, o_r):
        o_r[...] = (zr_r[:, 2:3] + h1_r[:, 0:1] * d1_r[...]
                    + h2_r[:, 0:1] * d2_r[...] + k_r[0, 0])

    return pl.pallas_call(
        body,
        grid=(grid,),
        in_specs=[_rows(8), _rows(8), _rows(1), _rows(1), _rows(3), _full((1, 1))],
        out_specs=_rows(1),
        out_shape=jax.ShapeDtypeStruct((npad, 1), jnp.float32),
    )(h1raw, h2raw, d1, d2, zr, kout)


def kernel(x, edge_index, edge_index_aux,
           W1, b1,
           Wk1_1, bk1_1, Wk1_2, bk1_2,
           Wk2_1, bk2_1, Wk2_2, bk2_2,
           Wk3_1, bk3_1, Wk3_2, bk3_2,
           Wr0, br0, Wr1, br1, Wr2, br2, Wr3, br3,
           w0, w1, w2, w3):
    n = x.shape[0]
    h = W1.shape[1]
    e = edge_index.shape[1]
    n_sub = 16

    rpt = -(-n // n_sub)
    rpt = -(-rpt // 8) * 8
    npad = rpt * n_sub
    chunk_total = -(-e // 128)
    cpt = -(-chunk_total // n_sub)
    cpt = -(-cpt // _NB) * _NB
    epad = cpt * n_sub * 128

    w0s, w1s, w2s, w3s = w0[0], w1[0], w2[0], w3[0]
    Wk1 = [Wk1_1, Wk1_2]
    bk1 = [bk1_1, bk1_2]
    Wk2 = [Wk2_1, Wk2_2]
    bk2 = [bk2_1, bk2_2]
    Wk3 = [Wk3_1, Wk3_2]
    bk3 = [bk3_1, bk3_2]
    Wr1s = [Wr1[:h, 0], Wr1[h:, 0]]
    Wr2s = [Wr2[:h, 0], Wr2[h:, 0]]
    Wr3s = [Wr3[:h, 0], Wr3[h:, 0]]

    g1 = [Wk1[j] @ Wr1s[j] for j in range(2)]
    u = [Wk2[j] @ Wr2s[j] for j in range(2)]
    us = [[u[j][:h], u[j][h:]] for j in range(2)]
    c2 = [[Wk1[m] @ us[j][m] for m in range(2)] for j in range(2)]
    v = [Wk3[j] @ Wr3s[j] for j in range(2)]
    vs = [[v[j][:h], v[j][h:]] for j in range(2)]
    t = [[Wk2[m] @ vs[j][m] for m in range(2)] for j in range(2)]
    ts = [[[t[j][m][:h], t[j][m][h:]] for m in range(2)] for j in range(2)]
    c3 = [[[Wk1[nn] @ ts[j][m][nn] for nn in range(2)] for m in range(2)]
          for j in range(2)]

    k_out = (w0s * br0[0] + w1s * br1[0] + w2s * br2[0] + w3s * br3[0]
             + w1s * sum(bk1[j] @ Wr1s[j] for j in range(2))
             + w2s * sum(bk2[j] @ Wr2s[j] for j in range(2))
             + w3s * sum(bk3[j] @ Wr3s[j] for j in range(2)))
    kap2 = [sum(bk1[m] @ us[j][m] for m in range(2)) for j in range(2)]
    kap3a = [sum(bk2[m] @ vs[j][m] for m in range(2)) for j in range(2)]
    kap3b = [[sum(bk1[nn] @ ts[j][m][nn] for nn in range(2)) for m in range(2)]
             for j in range(2)]

    cols = [w3s * c3[j][m][0] for j in range(2) for m in range(2)]
    cols += [w2s * c2[0][0], w2s * c2[1][0]]
    cols += [w3s * c3[j][m][1] for j in range(2) for m in range(2)]
    cols += [w2s * c2[0][1], w2s * c2[1][1]]
    cols += [w1s * g1[0], w1s * g1[1], w0s * Wr0[:, 0]]
    C = jnp.stack(cols, axis=1)

    consts2 = jnp.stack([w3s * kap3b[0][0], w3s * kap3b[0][1],
                         w3s * kap3b[1][0], w3s * kap3b[1][1],
                         w2s * kap2[0], w2s * kap2[1],
                         jnp.float32(0.0), jnp.float32(0.0)])[None, :]
    consts3 = jnp.stack([w3s * kap3a[0], w3s * kap3a[1]])[None, :]
    kout = k_out[None, None]

    def pad_edges(ei):
        p = jnp.full((2, epad - e), n, dtype=jnp.int32)
        return jnp.concatenate([ei, p], axis=1).reshape(2, epad // 128, 128)

    e1 = pad_edges(edge_index)
    e2 = pad_edges(edge_index_aux)

    deg1, deg2 = _sc_degree(e1, e2, npad, n_sub, cpt)
    deg1 = deg1.reshape(npad, 1)
    deg2 = deg2.reshape(npad, 1)

    t1, t2, zr, d1, d2 = _tc_dense(x, W1, b1[None, :], C, deg1, deg2, npad)

    y1raw, y2raw = _sc_stage(e1, e2, t1, t2, 8, npad, n_sub, cpt)

    u1, u2, s3p = _tc_prep2(y1raw, y2raw, d1, d2, zr, consts2, npad)

    q1raw, q2raw = _sc_stage(e1, e2, u1, u2, 8, npad, n_sub, cpt)

    g1t, g2t = _tc_prep3(q1raw, q2raw, d1, d2, s3p, consts3, npad)

    h1raw, h2raw = _sc_stage(e1, e2, g1t, g2t, 8, npad, n_sub, cpt)

    out = _tc_final(h1raw, h2raw, d1, d2, zr, kout, npad)
    return out[:n]


# speedup vs baseline: 44.0394x; 1.0596x over previous
import functools

import jax
import jax.numpy as jnp
from jax import lax
from jax.experimental import pallas as pl
from jax.experimental.pallas import tpu as pltpu
from jax.experimental.pallas import tpu_sc as plsc

_NB = 8
_ROW_BLK = 3136
_SC_PARAMS = pltpu.CompilerParams(use_tc_tiling_on_sc=False)


def _sc_degree(e1, e2, npad, n_sub, cpt):
    rpt = npad // n_sub
    nbg = cpt // _NB
    mesh = plsc.VectorSubcoreMesh(core_axis_name="c", subcore_axis_name="s")
    zeros = jnp.zeros((npad,), jnp.float32)

    @functools.partial(
        pl.kernel,
        out_type=[jax.ShapeDtypeStruct((npad,), jnp.float32)] * 2,
        mesh=mesh,
        compiler_params=_SC_PARAMS,
        scratch_types=[
            pltpu.VMEM_SHARED((npad,), jnp.float32),
            pltpu.VMEM((_NB, 128), jnp.int32),
            pltpu.VMEM((128,), jnp.float32),
            pltpu.VMEM((rpt,), jnp.float32),
            pltpu.SemaphoreType.DMA,
        ],
    )
    def knl(e1_h, e2_h, z_h, d1_h, d2_h, acc, dstb, onesb, vbuf, ssem):
        c = lax.axis_index("c")
        s = lax.axis_index("s")
        for i in range(8):
            onesb[pl.ds(i * 16, 16)] = jnp.full((16,), 1.0, jnp.float32)
        pltpu.sync_copy(z_h.at[pl.ds(s * rpt, rpt)], vbuf)
        pltpu.sync_copy(vbuf, acc.at[pl.ds(s * rpt, rpt)])
        plsc.subcore_barrier()

        def run(e_h, out_h):
            base = s * cpt

            def outer(g, carry):
                gb = base + g * _NB
                pltpu.sync_copy(e_h.at[1, pl.ds(gb, _NB)], dstb)
                cps = [pltpu.async_copy(onesb, acc.at[dstb.at[b]], ssem, add=True)
                       for b in range(_NB)]
                for cp in cps:
                    cp.wait()
                return carry

            lax.fori_loop(0, nbg, outer, 0)
            plsc.subcore_barrier()
            pltpu.sync_copy(acc.at[pl.ds(s * rpt, rpt)], vbuf)
            pltpu.sync_copy(vbuf, out_h.at[pl.ds(s * rpt, rpt)])

        @pl.when(c == 0)
        def _():
            run(e1_h, d1_h)

        @pl.when(c == 1)
        def _():
            run(e2_h, d2_h)

    return knl(e1, e2, zeros)


def _sc_stage(e1, e2, t1, t2, k, npad, n_sub, cpt):
    rpt = npad // n_sub
    nbg = cpt // _NB
    mesh = plsc.VectorSubcoreMesh(core_axis_name="c", subcore_axis_name="s")
    zeros = jnp.zeros((npad, k), jnp.float32)

    @functools.partial(
        pl.kernel,
        out_type=[jax.ShapeDtypeStruct((npad, k), jnp.float32)] * 2,
        mesh=mesh,
        compiler_params=_SC_PARAMS,
        scratch_types=[
            pltpu.VMEM_SHARED((npad, k), jnp.float32),
            pltpu.VMEM_SHARED((npad, k), jnp.float32),
            pltpu.VMEM((_NB, 128), jnp.int32),
            pltpu.VMEM((_NB, 128), jnp.int32),
            pltpu.VMEM((_NB, 128, k), jnp.float32),
            pltpu.VMEM((rpt, k), jnp.float32),
            pltpu.SemaphoreType.DMA,
            pltpu.SemaphoreType.DMA,
        ],
    )
    def knl(e1_h, e2_h, t1_h, t2_h, z_h, y1_h, y2_h,
            acc, tbl, srcb, dstb, rows, vbuf, gsem, ssem):
        c = lax.axis_index("c")
        s = lax.axis_index("s")

        def run(e_h, t_h, out_h):
            sl = pl.ds(s * rpt, rpt)
            pltpu.sync_copy(t_h.at[sl], vbuf)
            pltpu.sync_copy(vbuf, tbl.at[sl])
            pltpu.sync_copy(z_h.at[sl], vbuf)
            pltpu.sync_copy(vbuf, acc.at[sl])
            plsc.subcore_barrier()
            base = s * cpt

            def outer(g, carry):
                gb = base + g * _NB
                pltpu.sync_copy(e_h.at[0, pl.ds(gb, _NB)], srcb)
                pltpu.sync_copy(e_h.at[1, pl.ds(gb, _NB)], dstb)
                gcps = [pltpu.async_copy(tbl.at[srcb.at[b]], rows.at[b], gsem)
                        for b in range(_NB)]
                for cp in gcps:
                    cp.wait()
                scps = [pltpu.async_copy(rows.at[b], acc.at[dstb.at[b]], ssem, add=True)
                        for b in range(_NB)]
                for cp in scps:
                    cp.wait()
                return carry

            lax.fori_loop(0, nbg, outer, 0)
            plsc.subcore_barrier()
            pltpu.sync_copy(acc.at[sl], vbuf)
            pltpu.sync_copy(vbuf, out_h.at[sl])

        @pl.when(c == 0)
        def _():
            run(e1_h, t1_h, y1_h)

        @pl.when(c == 1)
        def _():
            run(e2_h, t2_h, y2_h)

    return knl(e1, e2, t1, t2, zeros)


def _full(shape):
    return pl.BlockSpec(shape, lambda i: (0,) * len(shape))


def _rows(k):
    return pl.BlockSpec((_ROW_BLK, k), lambda i: (i, 0))


def _tc_dense(x, W1, b1, C, deg1, deg2, npad):
    n = x.shape[0]
    d_in = x.shape[1]
    h = W1.shape[1]
    grid = (npad + _ROW_BLK - 1) // _ROW_BLK

    def body(x_r, w1_r, b1_r, c_r, g1_r, g2_r, t1_o, t2_o, zr_o, d1_o, d2_o):
        r0 = jnp.maximum(jnp.dot(x_r[...], w1_r[...],
                                 preferred_element_type=jnp.float32) + b1_r[...], 0.0)
        z = jnp.dot(r0, c_r[...], preferred_element_type=jnp.float32)
        d1 = jnp.where(g1_r[...] > 0,
                       lax.rsqrt(jnp.maximum(g1_r[...], 1e-12)), 0.0)
        d2 = jnp.where(g2_r[...] > 0,
                       lax.rsqrt(jnp.maximum(g2_r[...], 1e-12)), 0.0)
        pad = jnp.zeros((z.shape[0], 2), jnp.float32)
        t1_o[...] = jnp.concatenate([z[:, 0:6] * d1, pad], axis=1)
        t2_o[...] = jnp.concatenate([z[:, 6:12] * d2, pad], axis=1)
        zr_o[...] = z[:, 12:15]
        d1_o[...] = d1
        d2_o[...] = d2

    return pl.pallas_call(
        body,
        grid=(grid,),
        in_specs=[_rows(d_in), _full((d_in, h)), _full((1, h)), _full((h, 15)),
                  _rows(1), _rows(1)],
        out_specs=[_rows(8), _rows(8), _rows(3), _rows(1), _rows(1)],
        out_shape=[jax.ShapeDtypeStruct((npad, 8), jnp.float32),
                   jax.ShapeDtypeStruct((npad, 8), jnp.float32),
                   jax.ShapeDtypeStruct((npad, 3), jnp.float32),
                   jax.ShapeDtypeStruct((npad, 1), jnp.float32),
                   jax.ShapeDtypeStruct((npad, 1), jnp.float32)],
    )(x, W1, b1, C, deg1, deg2)


def _tc_prep2(y1raw, y2raw, d1, d2, zr, consts, npad):
    grid = (npad + _ROW_BLK - 1) // _ROW_BLK

    def body(y1_r, y2_r, d1_r, d2_r, zr_r, c_r, u1_o, u2_o, s3_o):
        y1 = y1_r[...] * d1_r[...]
        y2 = y2_r[...] * d2_r[...]
        p = y1[:, 0:4] + y2[:, 0:4] + c_r[0, 0:4]
        pad = jnp.zeros((p.shape[0], 6), jnp.float32)
        u1_o[...] = jnp.concatenate(
            [jnp.concatenate([p[:, 0:1], p[:, 2:3]], axis=1) * d1_r[...], pad], axis=1)
        u2_o[...] = jnp.concatenate(
            [jnp.concatenate([p[:, 1:2], p[:, 3:4]], axis=1) * d2_r[...], pad], axis=1)
        s3_o[...] = zr_r[:, 0:2] + y1[:, 4:6] + y2[:, 4:6] + c_r[0, 4:6]

    return pl.pallas_call(
        body,
        grid=(grid,),
        in_specs=[_rows(8), _rows(8), _rows(1), _rows(1), _rows(3), _full((1, 8))],
        out_specs=[_rows(8), _rows(8), _rows(2)],
        out_shape=[jax.ShapeDtypeStruct((npad, 8), jnp.float32),
                   jax.ShapeDtypeStruct((npad, 8), jnp.float32),
                   jax.ShapeDtypeStruct((npad, 2), jnp.float32)],
    )(y1raw, y2raw, d1, d2, zr, consts)


def _tc_prep3(q1raw, q2raw, d1, d2, s3p, consts, npad):
    grid = (npad + _ROW_BLK - 1) // _ROW_BLK

    def body(q1_r, q2_r, d1_r, d2_r, s3_r, c_r, g1_o, g2_o):
        f = (s3_r[...] + q1_r[:, 0:2] * d1_r[...] + q2_r[:, 0:2] * d2_r[...]
             + c_r[0, :])
        pad = jnp.zeros((f.shape[0], 7), jnp.float32)
        g1_o[...] = jnp.concatenate([f[:, 0:1] * d1_r[...], pad], axis=1)
        g2_o[...] = jnp.concatenate([f[:, 1:2] * d2_r[...], pad], axis=1)

    return pl.pallas_call(
        body,
        grid=(grid,),
        in_specs=[_rows(8), _rows(8), _rows(1), _rows(1), _rows(2), _full((1, 2))],
        out_specs=[_rows(8), _rows(8)],
        out_shape=[jax.ShapeDtypeStruct((npad, 8), jnp.float32)] * 2,
    )(q1raw, q2raw, d1, d2, s3p, consts)


def _tc_final(h1raw, h2raw, d1, d2, zr, kout, npad):
    grid = (npad + _ROW_BLK - 1) // _ROW_BLK

    def body(h1_r, h2_r, d1_r, d2_r, zr_r, k_r, o_r):
        o_r[...] = (zr_r[:, 2:3] + h1_r[:, 0:1] * d1_r[...]
                    + h2_r[:, 0:1] * d2_r[...] + k_r[0, 0])

    return pl.pallas_call(
        body,
        grid=(grid,),
        in_specs=[_rows(8), _rows(8), _rows(1), _rows(1), _rows(3), _full((1, 1))],
        out_specs=_rows(1),
        out_shape=jax.ShapeDtypeStruct((npad, 1), jnp.float32),
    )(h1raw, h2raw, d1, d2, zr, kout)


def kernel(x, edge_index, edge_index_aux,
           W1, b1,
           Wk1_1, bk1_1, Wk1_2, bk1_2,
           Wk2_1, bk2_1, Wk2_2, bk2_2,
           Wk3_1, bk3_1, Wk3_2, bk3_2,
           Wr0, br0, Wr1, br1, Wr2, br2, Wr3, br3,
           w0, w1, w2, w3):
    n = x.shape[0]
    h = W1.shape[1]
    e = edge_index.shape[1]
    n_sub = 16

    rpt = -(-n // n_sub)
    rpt = -(-rpt // 8) * 8
    npad = rpt * n_sub
    chunk_total = -(-e // 128)
    cpt = -(-chunk_total // n_sub)
    cpt = -(-cpt // _NB) * _NB
    epad = cpt * n_sub * 128

    w0s, w1s, w2s, w3s = w0[0], w1[0], w2[0], w3[0]
    Wk1 = [Wk1_1, Wk1_2]
    bk1 = [bk1_1, bk1_2]
    Wk2 = [Wk2_1, Wk2_2]
    bk2 = [bk2_1, bk2_2]
    Wk3 = [Wk3_1, Wk3_2]
    bk3 = [bk3_1, bk3_2]
    Wr1s = [Wr1[:h, 0], Wr1[h:, 0]]
    Wr2s = [Wr2[:h, 0], Wr2[h:, 0]]
    Wr3s = [Wr3[:h, 0], Wr3[h:, 0]]

    g1 = [Wk1[j] @ Wr1s[j] for j in range(2)]
    u = [Wk2[j] @ Wr2s[j] for j in range(2)]
    us = [[u[j][:h], u[j][h:]] for j in range(2)]
    c2 = [[Wk1[m] @ us[j][m] for m in range(2)] for j in range(2)]
    v = [Wk3[j] @ Wr3s[j] for j in range(2)]
    vs = [[v[j][:h], v[j][h:]] for j in range(2)]
    t = [[Wk2[m] @ vs[j][m] for m in range(2)] for j in range(2)]
    ts = [[[t[j][m][:h], t[j][m][h:]] for m in range(2)] for j in range(2)]
    c3 = [[[Wk1[nn] @ ts[j][m][nn] for nn in range(2)] for m in range(2)]
          for j in range(2)]

    k_out = (w0s * br0[0] + w1s * br1[0] + w2s * br2[0] + w3s * br3[0]
             + w1s * sum(bk1[j] @ Wr1s[j] for j in range(2))
             + w2s * sum(bk2[j] @ Wr2s[j] for j in range(2))
             + w3s * sum(bk3[j] @ Wr3s[j] for j in range(2)))
    kap2 = [sum(bk1[m] @ us[j][m] for m in range(2)) for j in range(2)]
    kap3a = [sum(bk2[m] @ vs[j][m] for m in range(2)) for j in range(2)]
    kap3b = [[sum(bk1[nn] @ ts[j][m][nn] for nn in range(2)) for m in range(2)]
             for j in range(2)]

    cols = [w3s * c3[j][m][0] for j in range(2) for m in range(2)]
    cols += [w2s * c2[0][0], w2s * c2[1][0]]
    cols += [w3s * c3[j][m][1] for j in range(2) for m in range(2)]
    cols += [w2s * c2[0][1], w2s * c2[1][1]]
    cols += [w1s * g1[0], w1s * g1[1], w0s * Wr0[:, 0]]
    C = jnp.stack(cols, axis=1)

    consts2 = jnp.stack([w3s * kap3b[0][0], w3s * kap3b[0][1],
                         w3s * kap3b[1][0], w3s * kap3b[1][1],
                         w2s * kap2[0], w2s * kap2[1],
                         jnp.float32(0.0), jnp.float32(0.0)])[None, :]
    consts3 = jnp.stack([w3s * kap3a[0], w3s * kap3a[1]])[None, :]
    kout = k_out[None, None]

    def pad_edges(ei):
        p = jnp.full((2, epad - e), n, dtype=jnp.int32)
        return jnp.concatenate([ei, p], axis=1).reshape(2, epad // 128, 128)

    e1 = pad_edges(edge_index)
    e2 = pad_edges(edge_index_aux)

    deg1, deg2 = _sc_degree(e1, e2, npad, n_sub, cpt)
    deg1 = deg1.reshape(npad, 1)
    deg2 = deg2.reshape(npad, 1)

    t1, t2, zr, d1, d2 = _tc_dense(x, W1, b1[None, :], C, deg1, deg2, npad)

    y1raw, y2raw = _sc_stage(e1, e2, t1, t2, 8, npad, n_sub, cpt)

    u1, u2, s3p = _tc_prep2(y1raw, y2raw, d1, d2, zr, consts2, npad)

    q1raw, q2raw = _sc_stage(e1, e2, u1, u2, 8, npad, n_sub, cpt)

    g1t, g2t = _tc_prep3(q1raw, q2raw, d1, d2, s3p, consts3, npad)

    h1raw, h2raw = _sc_stage(e1, e2, g1t, g2t, 8, npad, n_sub, cpt)

    out = _tc_final(h1raw, h2raw, d1, d2, zr, kout, npad)
    return out[:n]


# speedup vs baseline: 94.9441x; 2.1559x over previous
import functools

import jax
import jax.numpy as jnp
from jax import lax
from jax.experimental import pallas as pl
from jax.experimental.pallas import tpu as pltpu
from jax.experimental.pallas import tpu_sc as plsc

_NB = 16
_TC_BLK = 3200
_N_SUB = 16
_SC_PARAMS = pltpu.CompilerParams(use_tc_tiling_on_sc=False,
                                  needs_layout_passes=False)
_MESH = dict(core_axis_name="c", subcore_axis_name="s")
_PCH = 640


def _iota16():
    return lax.iota(jnp.int32, 16)


def _rsqrt16(v):
    bits = plsc.bitcast(v, jnp.int32)
    y = plsc.bitcast(jnp.int32(0x5F3759DF) - lax.shift_right_logical(bits, 1),
                     jnp.float32)
    h = v * 0.5
    for _ in range(4):
        y = y * (1.5 - h * y * y)
    return jnp.where(v > 0.0, y, 0.0)


def _sc_degree(e1, e2, npad, cpt):
    rpt = npad // _N_SUB
    nbg = cpt // _NB
    zeros = jnp.zeros((npad,), jnp.float32)

    @functools.partial(
        pl.kernel,
        out_type=[jax.ShapeDtypeStruct((npad,), jnp.float32)] * 2,
        mesh=plsc.VectorSubcoreMesh(**_MESH),
        compiler_params=_SC_PARAMS,
        scratch_types=[
            pltpu.VMEM_SHARED((npad,), jnp.float32),
            pltpu.VMEM((cpt // 4, 128), jnp.int32),
            pltpu.VMEM((128,), jnp.float32),
            pltpu.VMEM((rpt,), jnp.float32),
            pltpu.SemaphoreType.DMA,
        ],
    )
    def knl(e1_h, e2_h, z_h, d1_h, d2_h, acc, dstb, onesb, vbuf, ssem):
        c = lax.axis_index("c")
        s = lax.axis_index("s")
        for i in range(8):
            onesb[pl.ds(i * 16, 16)] = jnp.full((16,), 1.0, jnp.float32)
        sl = pl.ds(s * rpt, rpt)
        pltpu.sync_copy(z_h.at[sl], vbuf)
        pltpu.sync_copy(vbuf, acc.at[sl])
        plsc.subcore_barrier()

        def run(e_h, out_h):
            base = s * cpt
            half = cpt // 4

            for hf in range(4):
                pltpu.sync_copy(e_h.at[1, pl.ds(base + hf * half, half)], dstb)

                def outer(g, carry):
                    k0 = g * _NB
                    cps = [pltpu.async_copy(onesb, acc.at[dstb.at[k0 + b]],
                                            ssem, add=True)
                           for b in range(_NB)]
                    for cp in cps:
                        cp.wait()
                    return carry

                lax.fori_loop(0, half // _NB, outer, 0)
            plsc.subcore_barrier()
            pltpu.sync_copy(acc.at[sl], vbuf)

            def inv(i, carry):
                g = pl.ds(i * 16, 16)
                vbuf[g] = _rsqrt16(vbuf[g])
                return carry

            lax.fori_loop(0, rpt // 16, inv, 0)
            pltpu.sync_copy(vbuf, out_h.at[sl])

        @pl.when(c == 0)
        def _():
            run(e1_h, d1_h)

        @pl.when(c == 1)
        def _():
            run(e2_h, d2_h)

    return knl(e1, e2, zeros)


def _edge_pass(e_h, tbl, acc, srcb, dstb, rows, gsem, ssem, s, cpt, nbg):
    base = s * cpt
    half = cpt // 4

    for hf in range(4):
        pltpu.sync_copy(e_h.at[0, pl.ds(base + hf * half, half)], srcb)
        pltpu.sync_copy(e_h.at[1, pl.ds(base + hf * half, half)], dstb)

        def outer(g, carry):
            k0 = g * _NB
            gcps = [pltpu.async_copy(tbl.at[srcb.at[k0 + b]], rows.at[b], gsem)
                    for b in range(_NB)]
            scps = []
            for b in range(_NB):
                gcps[b].wait()
                scps.append(pltpu.async_copy(rows.at[b], acc.at[dstb.at[k0 + b]],
                                             ssem, add=True))
            for cp in scps:
                cp.wait()
            return carry

        lax.fori_loop(0, half // _NB, outer, 0)


def _stage_scaffold(npad, cpt, extra_scratch):
    rpt = npad // _N_SUB
    return dict(
        mesh=plsc.VectorSubcoreMesh(**_MESH),
        compiler_params=_SC_PARAMS,
        scratch_types=[
            pltpu.VMEM_SHARED((npad, 8), jnp.float32),
            pltpu.VMEM_SHARED((npad, 8), jnp.float32),
            pltpu.VMEM((cpt // 4, 128), jnp.int32),
            pltpu.VMEM((cpt // 4, 128), jnp.int32),
            pltpu.VMEM((_NB, 128, 8), jnp.float32),
            pltpu.VMEM((_PCH, 8), jnp.float32),
            pltpu.SemaphoreType.DMA,
            pltpu.SemaphoreType.DMA,
        ] + extra_scratch,
    )


def _sc_stage1(e1, e2, za3, zb3, d1, d2, zeros8, npad, cpt):
    rpt = npad // _N_SUB
    nbg = cpt // _NB
    nblk = rpt // 128

    @functools.partial(
        pl.kernel,
        out_type=[jax.ShapeDtypeStruct((npad, 8), jnp.float32)] * 2,
        **_stage_scaffold(npad, cpt, [
            pltpu.VMEM((_PCH // 128, 8, 128), jnp.float32),
            pltpu.VMEM((_PCH,), jnp.float32),
        ]),
    )
    def knl(e1_h, e2_h, za_h, zb_h, d1_h, d2_h, z8_h, y1_h, y2_h,
            acc, tbl, srcb, dstb, rows, tbuf, gsem, ssem, zbuf, dbuf):
        c = lax.axis_index("c")
        s = lax.axis_index("s")
        sl = pl.ds(s * rpt, rpt)

        def run(e_h, z_h, dv_h, out_h):
            for pch in range(rpt // _PCH):
                cs = pl.ds(s * rpt + pch * _PCH, _PCH)
                pltpu.sync_copy(
                    z_h.at[pl.ds(s * nblk + pch * (_PCH // 128), _PCH // 128)],
                    zbuf)
                pltpu.sync_copy(dv_h.at[cs], dbuf)

                def build(i, carry):
                    dv = dbuf[pl.ds(i * 16, 16)]
                    rowi = i * 16 + _iota16()
                    for f in range(8):
                        val = zbuf[i // 8, f, pl.ds((i % 8) * 16, 16)] * dv
                        plsc.store_scatter(
                            tbuf, [rowi, jnp.full((16,), f, jnp.int32)], val)
                    return carry

                lax.fori_loop(0, _PCH // 16, build, 0)
                pltpu.sync_copy(tbuf, tbl.at[cs])
                pltpu.sync_copy(z8_h.at[cs], tbuf)
                pltpu.sync_copy(tbuf, acc.at[cs])
            plsc.subcore_barrier()
            _edge_pass(e_h, tbl, acc, srcb, dstb, rows, gsem, ssem, s, cpt, nbg)
            plsc.subcore_barrier()
            for pch in range(rpt // _PCH):
                cs = pl.ds(s * rpt + pch * _PCH, _PCH)
                pltpu.sync_copy(acc.at[cs], tbuf)
                pltpu.sync_copy(tbuf, out_h.at[cs])

        @pl.when(c == 0)
        def _():
            run(e1_h, za_h, d1_h, y1_h)

        @pl.when(c == 1)
        def _():
            run(e2_h, zb_h, d2_h, y2_h)

    return knl(e1, e2, za3, zb3, d1, d2, zeros8)


def _col16(buf, i, col):
    return plsc.load_gather(
        buf, [i * 16 + _iota16(), jnp.full((16,), col, jnp.int32)])


def _sc_stage2(e1, e2, y1, y2, d1, d2, cb, zeros8, npad, cpt):
    rpt = npad // _N_SUB
    nbg = cpt // _NB

    @functools.partial(
        pl.kernel,
        out_type=[jax.ShapeDtypeStruct((npad, 8), jnp.float32)] * 2,
        **_stage_scaffold(npad, cpt, [
            pltpu.VMEM((_PCH, 8), jnp.float32),
            pltpu.VMEM((_PCH, 8), jnp.float32),
            pltpu.VMEM((_PCH,), jnp.float32),
            pltpu.VMEM((_PCH,), jnp.float32),
            pltpu.VMEM((16, 16), jnp.float32),
        ]),
    )
    def knl(e1_h, e2_h, y1_h, y2_h, d1_h, d2_h, cb_h, z8_h, q1_h, q2_h,
            acc, tbl, srcb, dstb, rows, tbuf, gsem, ssem,
            ybufA, ybufB, dbuf1, dbuf2, cbuf):
        c = lax.axis_index("c")
        s = lax.axis_index("s")
        pltpu.sync_copy(cb_h, cbuf)

        def run(e_h, out_h, m):
            dm = dbuf1 if m == 0 else dbuf2
            for pch in range(rpt // _PCH):
                cs = pl.ds(s * rpt + pch * _PCH, _PCH)
                pltpu.sync_copy(y1_h.at[cs], ybufA)
                pltpu.sync_copy(y2_h.at[cs], ybufB)
                pltpu.sync_copy(d1_h.at[cs], dbuf1)
                pltpu.sync_copy(d2_h.at[cs], dbuf2)

                def build(i, carry):
                    d1v = dbuf1[pl.ds(i * 16, 16)]
                    d2v = dbuf2[pl.ds(i * 16, 16)]
                    dmv = dm[pl.ds(i * 16, 16)]
                    rowi = i * 16 + _iota16()
                    zero = jnp.zeros((16,), jnp.float32)
                    for j in range(2):
                        jm = 2 * j + m
                        p = (d1v * _col16(ybufA, i, jm)
                             + d2v * _col16(ybufB, i, jm) + cbuf[jm])
                        plsc.store_scatter(
                            tbuf, [rowi, jnp.full((16,), j, jnp.int32)], dmv * p)
                    for f in range(2, 8):
                        plsc.store_scatter(
                            tbuf, [rowi, jnp.full((16,), f, jnp.int32)], zero)
                    return carry

                lax.fori_loop(0, _PCH // 16, build, 0)
                pltpu.sync_copy(tbuf, tbl.at[cs])
                pltpu.sync_copy(z8_h.at[cs], tbuf)
                pltpu.sync_copy(tbuf, acc.at[cs])
            plsc.subcore_barrier()
            _edge_pass(e_h, tbl, acc, srcb, dstb, rows, gsem, ssem, s, cpt, nbg)
            plsc.subcore_barrier()
            for pch in range(rpt // _PCH):
                cs = pl.ds(s * rpt + pch * _PCH, _PCH)
                pltpu.sync_copy(acc.at[cs], tbuf)
                pltpu.sync_copy(tbuf, out_h.at[cs])

        @pl.when(c == 0)
        def _():
            run(e1_h, q1_h, 0)

        @pl.when(c == 1)
        def _():
            run(e2_h, q2_h, 1)

    return knl(e1, e2, y1, y2, d1, d2, cb, zeros8)


def _sc_stage3(e1, e2, y1, y2, q1, q2, zb3, d1, d2, cb, zeros8, npad, cpt):
    rpt = npad // _N_SUB
    nbg = cpt // _NB
    nblk = rpt // 128

    @functools.partial(
        pl.kernel,
        out_type=[jax.ShapeDtypeStruct((npad, 8), jnp.float32)] * 2,
        **_stage_scaffold(npad, cpt, [
            pltpu.VMEM((_PCH, 8), jnp.float32),
            pltpu.VMEM((_PCH, 8), jnp.float32),
            pltpu.VMEM((_PCH, 8), jnp.float32),
            pltpu.VMEM((_PCH, 8), jnp.float32),
            pltpu.VMEM((_PCH // 128, 8, 128), jnp.float32),
            pltpu.VMEM((_PCH,), jnp.float32),
            pltpu.VMEM((_PCH,), jnp.float32),
            pltpu.VMEM((16, 16), jnp.float32),
        ]),
    )
    def knl(e1_h, e2_h, y1_h, y2_h, q1_h, q2_h, zb_h, d1_h, d2_h, cb_h, z8_h,
            h1_h, h2_h,
            acc, tbl, srcb, dstb, rows, tbuf, gsem, ssem,
            ybufA, ybufB, qbufA, qbufB, zbuf, dbuf1, dbuf2, cbuf):
        c = lax.axis_index("c")
        s = lax.axis_index("s")
        pltpu.sync_copy(cb_h, cbuf)

        def run(e_h, out_h, j):
            dj = dbuf1 if j == 0 else dbuf2
            for pch in range(rpt // _PCH):
                cs = pl.ds(s * rpt + pch * _PCH, _PCH)
                pltpu.sync_copy(y1_h.at[cs], ybufA)
                pltpu.sync_copy(y2_h.at[cs], ybufB)
                pltpu.sync_copy(q1_h.at[cs], qbufA)
                pltpu.sync_copy(q2_h.at[cs], qbufB)
                pltpu.sync_copy(
                    zb_h.at[pl.ds(s * nblk + pch * (_PCH // 128), _PCH // 128)],
                    zbuf)
                pltpu.sync_copy(d1_h.at[cs], dbuf1)
                pltpu.sync_copy(d2_h.at[cs], dbuf2)

                def build(i, carry):
                    d1v = dbuf1[pl.ds(i * 16, 16)]
                    d2v = dbuf2[pl.ds(i * 16, 16)]
                    djv = dj[pl.ds(i * 16, 16)]
                    rowi = i * 16 + _iota16()
                    zero = jnp.zeros((16,), jnp.float32)
                    z1j = zbuf[i // 8, 6 + j, pl.ds((i % 8) * 16, 16)]
                    s3p = (z1j + d1v * _col16(ybufA, i, 4 + j)
                           + d2v * _col16(ybufB, i, 4 + j) + cbuf[4 + j])
                    fv = (s3p + d1v * _col16(qbufA, i, j)
                          + d2v * _col16(qbufB, i, j) + cbuf[6 + j])
                    plsc.store_scatter(
                        tbuf, [rowi, jnp.zeros((16,), jnp.int32)], djv * fv)
                    for ff in range(1, 8):
                        plsc.store_scatter(
                            tbuf, [rowi, jnp.full((16,), ff, jnp.int32)], zero)
                    return carry

                lax.fori_loop(0, _PCH // 16, build, 0)
                pltpu.sync_copy(tbuf, tbl.at[cs])
                pltpu.sync_copy(z8_h.at[cs], tbuf)
                pltpu.sync_copy(tbuf, acc.at[cs])
            plsc.subcore_barrier()
            _edge_pass(e_h, tbl, acc, srcb, dstb, rows, gsem, ssem, s, cpt, nbg)
            plsc.subcore_barrier()
            for pch in range(rpt // _PCH):
                cs = pl.ds(s * rpt + pch * _PCH, _PCH)
                pltpu.sync_copy(acc.at[cs], tbuf)
                pltpu.sync_copy(tbuf, out_h.at[cs])

        @pl.when(c == 0)
        def _():
            run(e1_h, h1_h, 0)

        @pl.when(c == 1)
        def _():
            run(e2_h, h2_h, 1)

    return knl(e1, e2, y1, y2, q1, q2, zb3, d1, d2, cb, zeros8)


def _sc_final(h1, h2, za3, d1, d2, cb, npad):
    rpt = npad // _N_SUB
    nblk = rpt // 128

    @functools.partial(
        pl.kernel,
        out_type=jax.ShapeDtypeStruct((npad,), jnp.float32),
        mesh=plsc.VectorSubcoreMesh(**_MESH),
        compiler_params=_SC_PARAMS,
        scratch_types=[
            pltpu.VMEM((_PCH, 8), jnp.float32),
            pltpu.VMEM((_PCH, 8), jnp.float32),
            pltpu.VMEM((_PCH // 128, 8, 128), jnp.float32),
            pltpu.VMEM((_PCH,), jnp.float32),
            pltpu.VMEM((_PCH,), jnp.float32),
            pltpu.VMEM((_PCH,), jnp.float32),
            pltpu.VMEM((16, 16), jnp.float32),
        ],
    )
    def knl(h1_h, h2_h, za_h, d1_h, d2_h, cb_h, o_h,
            hbufA, hbufB, zbuf, dbuf1, dbuf2, obuf, cbuf):
        c = lax.axis_index("c")
        s = lax.axis_index("s")

        @pl.when(c == 0)
        def _():
            pltpu.sync_copy(cb_h, cbuf)
            for pch in range(rpt // _PCH):
                cs = pl.ds(s * rpt + pch * _PCH, _PCH)
                pltpu.sync_copy(h1_h.at[cs], hbufA)
                pltpu.sync_copy(h2_h.at[cs], hbufB)
                pltpu.sync_copy(
                    za_h.at[pl.ds(s * nblk + pch * (_PCH // 128), _PCH // 128)],
                    zbuf)
                pltpu.sync_copy(d1_h.at[cs], dbuf1)
                pltpu.sync_copy(d2_h.at[cs], dbuf2)

                def comb(i, carry):
                    d1v = dbuf1[pl.ds(i * 16, 16)]
                    d2v = dbuf2[pl.ds(i * 16, 16)]
                    base = zbuf[i // 8, 6, pl.ds((i % 8) * 16, 16)]
                    obuf[pl.ds(i * 16, 16)] = (
                        base + d1v * _col16(hbufA, i, 0)
                        + d2v * _col16(hbufB, i, 0) + cbuf[8])
                    return carry

                lax.fori_loop(0, _PCH // 16, comb, 0)
                pltpu.sync_copy(obuf, o_h.at[cs])

    return knl(h1, h2, za3, d1, d2, cb)


def _tc_dense(x, W1, b1, C16, npad):
    d_in = x.shape[1]
    h = W1.shape[1]
    grid = npad // _TC_BLK

    def body(x_r, w1_r, b1_r, c_r, za_o, zb_o):
        r0 = jnp.maximum(
            jnp.dot(x_r[...], w1_r[...],
                    preferred_element_type=jnp.float32) + b1_r[...], 0.0)
        zt = lax.dot_general(c_r[...], r0, (((0,), (1,)), ((), ())),
                             preferred_element_type=jnp.float32)
        za_o[...] = zt[0:8, :]
        zb_o[...] = zt[8:16, :]

    return pl.pallas_call(
        body,
        grid=(grid,),
        in_specs=[pl.BlockSpec((_TC_BLK, d_in), lambda i: (i, 0)),
                  pl.BlockSpec((d_in, h), lambda i: (0, 0)),
                  pl.BlockSpec((1, h), lambda i: (0, 0)),
                  pl.BlockSpec((h, 16), lambda i: (0, 0))],
        out_specs=[pl.BlockSpec((8, _TC_BLK), lambda i: (0, i)),
                   pl.BlockSpec((8, _TC_BLK), lambda i: (0, i))],
        out_shape=[jax.ShapeDtypeStruct((8, npad), jnp.float32)] * 2,
    )(x, W1, b1, C16)


def kernel(x, edge_index, edge_index_aux,
           W1, b1,
           Wk1_1, bk1_1, Wk1_2, bk1_2,
           Wk2_1, bk2_1, Wk2_2, bk2_2,
           Wk3_1, bk3_1, Wk3_2, bk3_2,
           Wr0, br0, Wr1, br1, Wr2, br2, Wr3, br3,
           w0, w1, w2, w3):
    n = x.shape[0]
    h = W1.shape[1]
    e = edge_index.shape[1]

    rpt = -(-n // _N_SUB)
    rpt = -(-rpt // 128) * 128
    npad = rpt * _N_SUB
    chunk_total = -(-e // 128)
    cpt = -(-chunk_total // _N_SUB)
    cpt = -(-cpt // _NB) * _NB
    epad = cpt * _N_SUB * 128

    w0s, w1s, w2s, w3s = w0[0], w1[0], w2[0], w3[0]
    Wk1 = [Wk1_1, Wk1_2]
    bk1 = [bk1_1, bk1_2]
    Wk2 = [Wk2_1, Wk2_2]
    bk2 = [bk2_1, bk2_2]
    Wk3 = [Wk3_1, Wk3_2]
    bk3 = [bk3_1, bk3_2]
    Wr1s = [Wr1[:h, 0], Wr1[h:, 0]]
    Wr2s = [Wr2[:h, 0], Wr2[h:, 0]]
    Wr3s = [Wr3[:h, 0], Wr3[h:, 0]]

    g1 = [Wk1[j] @ Wr1s[j] for j in range(2)]
    u = [Wk2[j] @ Wr2s[j] for j in range(2)]
    us = [[u[j][:h], u[j][h:]] for j in range(2)]
    c2 = [[Wk1[m] @ us[j][m] for m in range(2)] for j in range(2)]
    v = [Wk3[j] @ Wr3s[j] for j in range(2)]
    vs = [[v[j][:h], v[j][h:]] for j in range(2)]
    t = [[Wk2[m] @ vs[j][m] for m in range(2)] for j in range(2)]
    ts = [[[t[j][m][:h], t[j][m][h:]] for m in range(2)] for j in range(2)]
    c3 = [[[Wk1[nn] @ ts[j][m][nn] for nn in range(2)] for m in range(2)]
          for j in range(2)]

    k_out = (w0s * br0[0] + w1s * br1[0] + w2s * br2[0] + w3s * br3[0]
             + w1s * sum(bk1[j] @ Wr1s[j] for j in range(2))
             + w2s * sum(bk2[j] @ Wr2s[j] for j in range(2))
             + w3s * sum(bk3[j] @ Wr3s[j] for j in range(2)))
    kap2 = [sum(bk1[m] @ us[j][m] for m in range(2)) for j in range(2)]
    kap3a = [sum(bk2[m] @ vs[j][m] for m in range(2)) for j in range(2)]
    kap3b = [[sum(bk1[nn] @ ts[j][m][nn] for nn in range(2)) for m in range(2)]
             for j in range(2)]

    zcol = jnp.zeros((h,), jnp.float32)
    cols = [w3s * c3[j][m][0] for j in range(2) for m in range(2)]
    cols += [w2s * c2[0][0], w2s * c2[1][0], w0s * Wr0[:, 0], zcol]
    cols += [w3s * c3[j][m][1] for j in range(2) for m in range(2)]
    cols += [w2s * c2[0][1], w2s * c2[1][1], w1s * g1[0], w1s * g1[1]]
    C16 = jnp.stack(cols, axis=1)

    cvals = jnp.stack([w3s * kap3b[0][0], w3s * kap3b[0][1],
                       w3s * kap3b[1][0], w3s * kap3b[1][1],
                       w2s * kap2[0], w2s * kap2[1],
                       w3s * kap3a[0], w3s * kap3a[1],
                       k_out] + [jnp.float32(0.0)] * 7)
    cb = jnp.tile(cvals[:, None], (1, 16))

    def pad_edges(ei):
        p = jnp.full((2, epad - e), n, dtype=jnp.int32)
        return jnp.concatenate([ei, p], axis=1).reshape(2, epad // 128, 128)

    e1 = pad_edges(edge_index)
    e2 = pad_edges(edge_index_aux)
    zeros8 = jnp.zeros((npad, 8), jnp.float32)

    d1, d2 = _sc_degree(e1, e2, npad, cpt)

    za, zb = _tc_dense(x, W1, b1[None, :], C16, npad)
    za3 = za.reshape(8, npad // 128, 128).transpose(1, 0, 2)
    zb3 = zb.reshape(8, npad // 128, 128).transpose(1, 0, 2)

    y1, y2 = _sc_stage1(e1, e2, za3, zb3, d1, d2, zeros8, npad, cpt)
    q1, q2 = _sc_stage2(e1, e2, y1, y2, d1, d2, cb, zeros8, npad, cpt)
    h1, h2 = _sc_stage3(e1, e2, y1, y2, q1, q2, zb3, d1, d2, cb, zeros8,
                        npad, cpt)

    out = _sc_final(h1, h2, za3, d1, d2, cb, npad)
    return out[:n, None]
